# baseline (speedup 1.0000x reference)
import functools

import jax
import jax.numpy as jnp
from jax import lax
from jax.experimental import pallas as pl
from jax.experimental.pallas import tpu as pltpu
from jax.experimental.pallas import tpu_sc as plsc

N = 10000
D = 128
E = 320000
NC = 2
NS = 16
NW = NC * NS
K = 128
NCH = 80
EPW = NCH * K
EPAD = NW * EPW
ROWS_PER_SUB = 632
NP = ROWS_PER_SUB * NS
NBUF = 2
L = 16

_mesh = plsc.VectorSubcoreMesh(core_axis_name="c", subcore_axis_name="s")


def _matmul_body(x_ref, w_ref, o_ref):
    o_ref[...] = jnp.dot(x_ref[...], w_ref[...],
                         preferred_element_type=jnp.float32)


def _matmul(x, W):
    Bn = 2000
    return pl.pallas_call(
        _matmul_body,
        grid=(N // Bn,),
        in_specs=[pl.BlockSpec((Bn, D), lambda i: (i, 0)),
                  pl.BlockSpec((D, D), lambda i: (0, 0))],
        out_specs=pl.BlockSpec((Bn, D), lambda i: (i, 0)),
        out_shape=jax.ShapeDtypeStruct((N, D), jnp.float32),
    )(x, W)


@functools.partial(
    pl.kernel,
    out_type=jax.ShapeDtypeStruct((NC, NP, 16), jnp.float32),
    mesh=_mesh,
    scratch_types=[
        pltpu.VMEM((NCH, 1, K), jnp.int32),
        pltpu.VMEM((K, 16), jnp.float32),
        [pltpu.VMEM((1, K), jnp.int32)] * NBUF,
        pltpu.VMEM_SHARED((NP, 16), jnp.float32),
        [pltpu.SemaphoreType.DMA] * NBUF,
    ],
    compiler_params=pltpu.CompilerParams(use_tc_tiling_on_sc=False),
)
def _deg_kernel(packed_hbm, zeros_hbm, ones_hbm, out_hbm,
                packed_v, ones_v, dstc, acc_sh, sems):
    cid = lax.axis_index("c")
    sid = lax.axis_index("s")
    wid = cid * NS + sid
    row0 = sid * ROWS_PER_SUB
    pltpu.sync_copy(zeros_hbm.at[pl.ds(row0, ROWS_PER_SUB)],
                    acc_sh.at[pl.ds(row0, ROWS_PER_SUB)])
    pltpu.sync_copy(ones_hbm, ones_v)
    pltpu.sync_copy(packed_hbm.at[wid], packed_v)
    plsc.subcore_barrier()

    def unpack_dst(j, buf):
        for i in range(K // L):
            v = packed_v[j, 0, pl.ds(i * L, L)]
            buf[0, pl.ds(i * L, L)] = v >> 16

    @pl.loop(0, NCH, step=NBUF)
    def _(c):
        for b in range(NBUF):
            @pl.when(c > 0)
            def _():
                pltpu.make_async_copy(
                    ones_v, acc_sh.at[dstc[b].at[0]], sems[b]).wait()

            unpack_dst(c + b, dstc[b])
            pltpu.async_copy(ones_v, acc_sh.at[dstc[b].at[0]],
                             sems[b], add=True)

    for b in range(NBUF):
        pltpu.make_async_copy(ones_v, acc_sh.at[dstc[b].at[0]],
                              sems[b]).wait()

    plsc.subcore_barrier()
    pltpu.sync_copy(acc_sh.at[pl.ds(row0, ROWS_PER_SUB)],
                    out_hbm.at[cid, pl.ds(row0, ROWS_PER_SUB)])


def _scale_body(xw_ref, degp_ref, o_ref):
    deg = 1.0 + degp_ref[0, :, 0:1] + degp_ref[1, :, 0:1]
    o_ref[...] = xw_ref[...] / jnp.sqrt(deg)


def _scale(xw, degp):
    Bn = 2000
    return pl.pallas_call(
        _scale_body,
        grid=(N // Bn,),
        in_specs=[pl.BlockSpec((Bn, D), lambda i: (i, 0)),
                  pl.BlockSpec((NC, Bn, 16), lambda i: (0, i, 0))],
        out_specs=pl.BlockSpec((Bn, D), lambda i: (i, 0)),
        out_shape=jax.ShapeDtypeStruct((N, D), jnp.float32),
    )(xw, degp)


@functools.partial(
    pl.kernel,
    out_type=jax.ShapeDtypeStruct((NC, NP, D), jnp.float32),
    mesh=_mesh,
    scratch_types=[
        pltpu.VMEM((NCH, 1, K), jnp.int32),
        [pltpu.VMEM((1, K), jnp.int32)] * NBUF,
        [pltpu.VMEM((1, K), jnp.int32)] * NBUF,
        [pltpu.VMEM((K, D), jnp.float32)] * NBUF,
        pltpu.VMEM_SHARED((NP, D), jnp.float32),
        [pltpu.SemaphoreType.DMA] * NBUF,
        [pltpu.SemaphoreType.DMA] * NBUF,
    ],
)
def _edge_kernel(table_hbm, packed_hbm, zeros_hbm, out_hbm,
                 packed_v, srcc, dstc, rows, acc_sh, gsems, ssems):
    cid = lax.axis_index("c")
    sid = lax.axis_index("s")
    wid = cid * NS + sid
    row0 = sid * ROWS_PER_SUB
    pltpu.sync_copy(zeros_hbm.at[pl.ds(row0, ROWS_PER_SUB)],
                    acc_sh.at[pl.ds(row0, ROWS_PER_SUB)])
    pltpu.sync_copy(packed_hbm.at[wid], packed_v)
    plsc.subcore_barrier()

    def unpack(j, sbuf, dbuf):
        for i in range(K // L):
            v = packed_v[j, 0, pl.ds(i * L, L)]
            sbuf[0, pl.ds(i * L, L)] = v & 0xFFFF
            dbuf[0, pl.ds(i * L, L)] = v >> 16

    for b in range(NBUF):
        unpack(b, srcc[b], dstc[b])
        pltpu.async_copy(table_hbm.at[srcc[b].at[0]], rows[b], gsems[b])

    @pl.loop(0, NCH, step=NBUF)
    def _(c):
        for b in range(NBUF):
            pltpu.make_async_copy(table_hbm.at[srcc[b].at[0]],
                                  rows[b], gsems[b]).wait()
            pltpu.async_copy(rows[b], acc_sh.at[dstc[b].at[0]],
                             ssems[b], add=True)
            pltpu.make_async_copy(rows[b], acc_sh.at[dstc[b].at[0]],
                                  ssems[b]).wait()

            @pl.when(c + b + NBUF < NCH)
            def _():
                unpack(c + b + NBUF, srcc[b], dstc[b])
                pltpu.async_copy(table_hbm.at[srcc[b].at[0]],
                                 rows[b], gsems[b])

    plsc.subcore_barrier()
    pltpu.sync_copy(acc_sh.at[pl.ds(row0, ROWS_PER_SUB)],
                    out_hbm.at[cid, pl.ds(row0, ROWS_PER_SUB)])


def _final_body(s_ref, xw_ref, degp_ref, b_ref, a_ref, o_ref):
    deg = 1.0 + degp_ref[0, :, 0:1] + degp_ref[1, :, 0:1]
    dinv = 1.0 / jnp.sqrt(deg)
    s = (s_ref[0] + s_ref[1]) * dinv + xw_ref[...] * (dinv * dinv) + b_ref[...]
    o_ref[...] = jnp.where(s > 0, s, a_ref[...] * s)


def _final(sums, xw, degp, b2, a2):
    Bn = 2000
    return pl.pallas_call(
        _final_body,
        grid=(N // Bn,),
        in_specs=[pl.BlockSpec((NC, Bn, D), lambda i: (0, i, 0)),
                  pl.BlockSpec((Bn, D), lambda i: (i, 0)),
                  pl.BlockSpec((NC, Bn, 16), lambda i: (0, i, 0)),
                  pl.BlockSpec((1, D), lambda i: (0, 0)),
                  pl.BlockSpec((1, D), lambda i: (0, 0))],
        out_specs=pl.BlockSpec((Bn, D), lambda i: (i, 0)),
        out_shape=jax.ShapeDtypeStruct((N, D), jnp.float32),
    )(sums, xw, degp, b2, a2)


def kernel(x, edge_index, W, b, prelu_a):
    src = edge_index[0].astype(jnp.int32)
    dst = edge_index[1].astype(jnp.int32)
    src_p = jnp.concatenate([src, jnp.zeros((EPAD - E,), jnp.int32)])
    dst_p = jnp.concatenate([dst, jnp.full((EPAD - E,), NP - 1, jnp.int32)])
    packed = (src_p | (dst_p << 16)).reshape(NW, NCH, 1, K)
    xw = _matmul(x, W)
    zeros_n16 = jnp.zeros((NP, 16), jnp.float32)
    zeros_nd = jnp.zeros((NP, D), jnp.float32)
    ones_k16 = jnp.ones((K, 16), jnp.float32)
    degp = _deg_kernel(packed, zeros_n16, ones_k16)
    scaled = _scale(xw, degp)
    sums = _edge_kernel(scaled, packed, zeros_nd)
    return _final(sums, xw, degp, b.reshape(1, D), prelu_a.reshape(1, D))
